# baseline (speedup 1.0000x reference)
import functools

import jax
import jax.numpy as jnp
from jax import lax
from jax.experimental import pallas as pl
from jax.experimental.pallas import tpu as pltpu
from jax.experimental.pallas import tpu_sc as plsc

_NUM_ENTITIES = 1000000
_EMBED_DIM = 64
_BATCH = 16384

_INFO = plsc.get_sparse_core_info()
_NC, _NS, _L = _INFO.num_cores, _INFO.num_subcores, _INFO.num_lanes
_NW = _NC * _NS
_BPW = _BATCH // _NW
_CHUNK = 128
_NCHUNK = _BPW // _CHUNK


def _rsqrt_vec(x):
    i = plsc.bitcast(x, jnp.int32)
    i = jnp.int32(0x5F3759DF) - (i >> 1)
    y = plsc.bitcast(i, jnp.float32)
    hx = x * jnp.float32(-0.5)
    c = jnp.float32(1.5)
    y = y * (c + hx * y * y)
    y = y * (c + hx * y * y)
    y = y * (c + hx * y * y)
    return y


def _sc_body(s_hbm, r_hbm, o_hbm, e_hbm, rt_hbm, out_hbm,
             idx_s, idx_r, idx_o, se_v, re_v, oe_v, out_v, sem):
    wid = lax.axis_index("s") * _NC + lax.axis_index("c")
    base = wid * _BPW

    pltpu.sync_copy(s_hbm.at[wid], idx_s)
    pltpu.sync_copy(r_hbm.at[wid], idx_r)
    pltpu.sync_copy(o_hbm.at[wid], idx_o)

    copies = []
    for j in range(_NCHUNK):
        dst = pl.ds(j * _CHUNK, _CHUNK)
        copies.append(pltpu.async_copy(e_hbm.at[idx_s.at[j]], se_v.at[dst], sem))
        copies.append(pltpu.async_copy(e_hbm.at[idx_o.at[j]], oe_v.at[dst], sem))
        copies.append(pltpu.async_copy(rt_hbm.at[idx_r.at[j]], re_v.at[dst], sem))
    for c in copies:
        c.wait()

    eps = jnp.float32(1e-24)
    iota = lax.iota(jnp.int32, _L)
    zero = jnp.zeros((_L,), jnp.float32)

    def group(g, carry):
        ridx = g * _L + iota
        ss, rs, os_ = zero, zero, zero
        for j in range(_EMBED_DIM):
            cj = jnp.full((_L,), j, jnp.int32)
            vs = plsc.load_gather(se_v, [ridx, cj])
            vr = plsc.load_gather(re_v, [ridx, cj])
            vo = plsc.load_gather(oe_v, [ridx, cj])
            ss = ss + vs * vs
            rs = rs + vr * vr
            os_ = os_ + vo * vo

        inv_s = _rsqrt_vec(jnp.maximum(ss, eps))
        inv_r = _rsqrt_vec(jnp.maximum(rs, eps))
        inv_o = _rsqrt_vec(jnp.maximum(os_, eps))

        acc = zero
        for j in range(_EMBED_DIM):
            cj = jnp.full((_L,), j, jnp.int32)
            vs = plsc.load_gather(se_v, [ridx, cj])
            vr = plsc.load_gather(re_v, [ridx, cj])
            vo = plsc.load_gather(oe_v, [ridx, cj])
            acc = acc + jnp.abs(vs * inv_s + vr * inv_r - vo * inv_o)
        out_v[pl.ds(g * _L, _L)] = acc
        return carry

    lax.fori_loop(0, _BPW // _L, group, 0)

    pltpu.sync_copy(out_v, out_hbm.at[pl.ds(base, _BPW)])


@jax.jit
def kernel(s, r, o, e_table, r_table):
    s3 = s.astype(jnp.int32).reshape(_NW, _NCHUNK, _CHUNK)
    r3 = r.astype(jnp.int32).reshape(_NW, _NCHUNK, _CHUNK)
    o3 = o.astype(jnp.int32).reshape(_NW, _NCHUNK, _CHUNK)

    mesh = plsc.VectorSubcoreMesh(core_axis_name="c", subcore_axis_name="s")
    run = functools.partial(
        pl.kernel,
        mesh=mesh,
        compiler_params=pltpu.CompilerParams(
            needs_layout_passes=False, use_tc_tiling_on_sc=False),
        out_type=jax.ShapeDtypeStruct((_BATCH,), jnp.float32),
        scratch_types=[
            pltpu.VMEM((_NCHUNK, _CHUNK), jnp.int32),
            pltpu.VMEM((_NCHUNK, _CHUNK), jnp.int32),
            pltpu.VMEM((_NCHUNK, _CHUNK), jnp.int32),
            pltpu.VMEM((_BPW, _EMBED_DIM), jnp.float32),
            pltpu.VMEM((_BPW, _EMBED_DIM), jnp.float32),
            pltpu.VMEM((_BPW, _EMBED_DIM), jnp.float32),
            pltpu.VMEM((_BPW,), jnp.float32),
            pltpu.SemaphoreType.DMA,
        ],
    )(_sc_body)
    return run(s3, r3, o3, e_table, r_table)


# speedup vs baseline: 1.0419x; 1.0419x over previous
import functools

import jax
import jax.numpy as jnp
from jax import lax
from jax.experimental import pallas as pl
from jax.experimental.pallas import tpu as pltpu
from jax.experimental.pallas import tpu_sc as plsc

_EMBED_DIM = 64
_BATCH = 16384

_INFO = plsc.get_sparse_core_info()
_NC, _NS, _L = _INFO.num_cores, _INFO.num_subcores, _INFO.num_lanes
_NW = _NC * _NS
_BPW = _BATCH // _NW
_CHUNK = 128
_PHASE = 256
_NCHUNK = _PHASE // _CHUNK


def _rsqrt_vec(x):
    i = plsc.bitcast(x, jnp.int32)
    i = jnp.int32(0x5F3759DF) - (i >> 1)
    y = plsc.bitcast(i, jnp.float32)
    hx = x * jnp.float32(-0.5)
    c = jnp.float32(1.5)
    y = y * (c + hx * y * y)
    y = y * (c + hx * y * y)
    y = y * (c + hx * y * y)
    return y


def _sc_body(s_hbm, r_hbm, o_hbm, e_hbm, rt_hbm, out_hbm,
             tmp_v, half_s, half_r, half_o, sel_s, sel_r, sel_o,
             se_v, re_v, oe_v, out_v, sem):
    wid = lax.axis_index("s") * _NC + lax.axis_index("c")
    base = wid * _BPW

    eps = jnp.float32(1e-24)
    iota = lax.iota(jnp.int32, _L)
    zero = jnp.zeros((_L,), jnp.float32)
    one = jnp.full((_L,), 1, jnp.int32)
    half_off = jnp.full((_L,), _EMBED_DIM, jnp.int32)

    for ph in range(_BPW // _PHASE):
        pbase = base + ph * _PHASE

        for hbm, half, sel in ((s_hbm, half_s, sel_s),
                               (r_hbm, half_r, sel_r),
                               (o_hbm, half_o, sel_o)):
            pltpu.sync_copy(hbm.at[pl.ds(pbase, _PHASE)], tmp_v)

            def split_body(k, carry):
                sl = pl.ds(k * _L, _L)
                v = tmp_v[sl]
                half[sl] = v >> 1
                sel[sl] = (v & one) * half_off
                return carry

            lax.fori_loop(0, _PHASE // _L, split_body, 0, unroll=4)

        copies = []
        for j in range(_NCHUNK):
            src = pl.ds(j * _CHUNK, _CHUNK)
            dst = pl.ds(j * _CHUNK, _CHUNK)
            copies.append(pltpu.async_copy(e_hbm.at[half_s.at[src]],
                                           se_v.at[dst], sem))
            copies.append(pltpu.async_copy(e_hbm.at[half_o.at[src]],
                                           oe_v.at[dst], sem))
            copies.append(pltpu.async_copy(rt_hbm.at[half_r.at[src]],
                                           re_v.at[dst], sem))
        for c in copies:
            c.wait()

        def group(g, carry):
            ridx = g * _L + iota
            gsl = pl.ds(g * _L, _L)
            ssel = sel_s[gsl]
            rsel = sel_r[gsl]
            osel = sel_o[gsl]

            def norm_body(j, c):
                ss, rs, os_ = c
                cj = (iota + j) & (_EMBED_DIM - 1)
                vs = plsc.load_gather(se_v, [ridx, ssel + cj])
                vr = plsc.load_gather(re_v, [ridx, rsel + cj])
                vo = plsc.load_gather(oe_v, [ridx, osel + cj])
                return (ss + vs * vs, rs + vr * vr, os_ + vo * vo)

            ss, rs, os_ = lax.fori_loop(0, _EMBED_DIM, norm_body,
                                        (zero, zero, zero), unroll=8)

            inv_s = _rsqrt_vec(jnp.maximum(ss, eps))
            inv_r = _rsqrt_vec(jnp.maximum(rs, eps))
            inv_o = _rsqrt_vec(jnp.maximum(os_, eps))

            def score_body(j, acc):
                cj = (iota + j) & (_EMBED_DIM - 1)
                vs = plsc.load_gather(se_v, [ridx, ssel + cj])
                vr = plsc.load_gather(re_v, [ridx, rsel + cj])
                vo = plsc.load_gather(oe_v, [ridx, osel + cj])
                return acc + jnp.abs(vs * inv_s + vr * inv_r - vo * inv_o)

            acc = lax.fori_loop(0, _EMBED_DIM, score_body, zero, unroll=8)
            out_v[gsl] = acc
            return carry

        lax.fori_loop(0, _PHASE // _L, group, 0)

        pltpu.sync_copy(out_v, out_hbm.at[pl.ds(pbase, _PHASE)])


@jax.jit
def kernel(s, r, o, e_table, r_table):
    n_e, n_r = e_table.shape[0], r_table.shape[0]
    e2 = e_table.reshape(n_e // 2, 2 * _EMBED_DIM)
    rt2 = r_table.reshape(n_r // 2, 2 * _EMBED_DIM)
    s1 = s.astype(jnp.int32)
    r1 = r.astype(jnp.int32)
    o1 = o.astype(jnp.int32)

    mesh = plsc.VectorSubcoreMesh(core_axis_name="c", subcore_axis_name="s")
    run = functools.partial(
        pl.kernel,
        mesh=mesh,
        compiler_params=pltpu.CompilerParams(needs_layout_passes=False),
        out_type=jax.ShapeDtypeStruct((_BATCH,), jnp.float32),
        scratch_types=[
            pltpu.VMEM((_PHASE,), jnp.int32),
            pltpu.VMEM((_PHASE,), jnp.int32),
            pltpu.VMEM((_PHASE,), jnp.int32),
            pltpu.VMEM((_PHASE,), jnp.int32),
            pltpu.VMEM((_PHASE,), jnp.int32),
            pltpu.VMEM((_PHASE,), jnp.int32),
            pltpu.VMEM((_PHASE,), jnp.int32),
            pltpu.VMEM((_PHASE, 2 * _EMBED_DIM), jnp.float32),
            pltpu.VMEM((_PHASE, 2 * _EMBED_DIM), jnp.float32),
            pltpu.VMEM((_PHASE, 2 * _EMBED_DIM), jnp.float32),
            pltpu.VMEM((_PHASE,), jnp.float32),
            pltpu.SemaphoreType.DMA,
        ],
    )(_sc_body)
    return run(s1, r1, o1, e2, rt2)


# speedup vs baseline: 1.4310x; 1.3734x over previous
import functools

import jax
import jax.numpy as jnp
from jax import lax
from jax.experimental import pallas as pl
from jax.experimental.pallas import tpu as pltpu
from jax.experimental.pallas import tpu_sc as plsc

_EMBED_DIM = 64
_BATCH = 16384
_GRP = 8

_INFO = plsc.get_sparse_core_info()
_NC, _NS, _L = _INFO.num_cores, _INFO.num_subcores, _INFO.num_lanes
_NW = _NC * _NS
_BPW = _BATCH // _NW
_PHASE = 32
_NG = _PHASE // _L


def _rsqrt_vec(x):
    i = plsc.bitcast(x, jnp.int32)
    i = jnp.int32(0x5F3759DF) - (i >> 1)
    y = plsc.bitcast(i, jnp.float32)
    hx = x * jnp.float32(-0.5)
    c = jnp.float32(1.5)
    y = y * (c + hx * y * y)
    y = y * (c + hx * y * y)
    y = y * (c + hx * y * y)
    return y


def _sc_body(s_hbm, r_hbm, o_hbm, e_hbm, rt_hbm, out_hbm,
             idx_s, idx_r, idx_o, se_v, re_v, oe_v, out_v, sem):
    wid = lax.axis_index("s") * _NC + lax.axis_index("c")
    base = wid * _BPW

    eps = jnp.float32(1e-24)
    iota = lax.iota(jnp.int32, _L)
    zero = jnp.zeros((_L,), jnp.float32)
    seven = jnp.full((_L,), _GRP - 1, jnp.int32)

    for ph in range(_BPW // _PHASE):
        pbase = base + ph * _PHASE

        pltpu.sync_copy(s_hbm.at[pl.ds(pbase, _PHASE)], idx_s)
        pltpu.sync_copy(r_hbm.at[pl.ds(pbase, _PHASE)], idx_r)
        pltpu.sync_copy(o_hbm.at[pl.ds(pbase, _PHASE)], idx_o)

        def fetch(k, carry):
            v_s = idx_s[pl.ds(k * _L, _L)]
            v_r = idx_r[pl.ds(k * _L, _L)]
            v_o = idx_o[pl.ds(k * _L, _L)]
            for t in range(_L):
                slot = k * _L + t
                dst = pl.ds(pl.multiple_of(slot * _GRP, _GRP), _GRP)
                src_s = pl.ds(pl.multiple_of((v_s[t] >> 3) * _GRP, _GRP), _GRP)
                src_r = pl.ds(pl.multiple_of((v_r[t] >> 3) * _GRP, _GRP), _GRP)
                src_o = pl.ds(pl.multiple_of((v_o[t] >> 3) * _GRP, _GRP), _GRP)
                pltpu.async_copy(e_hbm.at[src_s, :], se_v.at[dst, :], sem)
                pltpu.async_copy(rt_hbm.at[src_r, :], re_v.at[dst, :], sem)
                pltpu.async_copy(e_hbm.at[src_o, :], oe_v.at[dst, :], sem)
            return carry

        lax.fori_loop(0, _NG, fetch, 0)

        def drain(k, carry):
            for _ in range(3):
                pltpu.make_async_copy(
                    e_hbm.at[pl.ds(0, _GRP), :],
                    se_v.at[pl.ds(0, _GRP), :], sem).wait()
            return carry

        lax.fori_loop(0, _PHASE, drain, 0)

        def group(g, carry):
            gsl = pl.ds(g * _L, _L)
            slot16 = (g * _L + iota) * _GRP
            rl_s = slot16 + (idx_s[gsl] & seven)
            rl_r = slot16 + (idx_r[gsl] & seven)
            rl_o = slot16 + (idx_o[gsl] & seven)

            def norm_body(j, c):
                ss, rs, os_ = c
                cj = (iota + j) & (_EMBED_DIM - 1)
                vs = plsc.load_gather(se_v, [rl_s, cj])
                vr = plsc.load_gather(re_v, [rl_r, cj])
                vo = plsc.load_gather(oe_v, [rl_o, cj])
                return (ss + vs * vs, rs + vr * vr, os_ + vo * vo)

            ss, rs, os_ = lax.fori_loop(0, _EMBED_DIM, norm_body,
                                        (zero, zero, zero), unroll=8)

            inv_s = _rsqrt_vec(jnp.maximum(ss, eps))
            inv_r = _rsqrt_vec(jnp.maximum(rs, eps))
            inv_o = _rsqrt_vec(jnp.maximum(os_, eps))

            def score_body(j, acc):
                cj = (iota + j) & (_EMBED_DIM - 1)
                vs = plsc.load_gather(se_v, [rl_s, cj])
                vr = plsc.load_gather(re_v, [rl_r, cj])
                vo = plsc.load_gather(oe_v, [rl_o, cj])
                return acc + jnp.abs(vs * inv_s + vr * inv_r - vo * inv_o)

            acc = lax.fori_loop(0, _EMBED_DIM, score_body, zero, unroll=8)
            out_v[gsl] = acc
            return carry

        lax.fori_loop(0, _NG, group, 0)

        pltpu.sync_copy(out_v, out_hbm.at[pl.ds(pbase, _PHASE)])


@jax.jit
def kernel(s, r, o, e_table, r_table):
    s1 = s.astype(jnp.int32)
    r1 = r.astype(jnp.int32)
    o1 = o.astype(jnp.int32)

    mesh = plsc.VectorSubcoreMesh(core_axis_name="c", subcore_axis_name="s")
    run = functools.partial(
        pl.kernel,
        mesh=mesh,
        compiler_params=pltpu.CompilerParams(needs_layout_passes=False),
        out_type=jax.ShapeDtypeStruct((_BATCH,), jnp.float32),
        scratch_types=[
            pltpu.VMEM((_PHASE,), jnp.int32),
            pltpu.VMEM((_PHASE,), jnp.int32),
            pltpu.VMEM((_PHASE,), jnp.int32),
            pltpu.VMEM((_PHASE * _GRP, _EMBED_DIM), jnp.float32),
            pltpu.VMEM((_PHASE * _GRP, _EMBED_DIM), jnp.float32),
            pltpu.VMEM((_PHASE * _GRP, _EMBED_DIM), jnp.float32),
            pltpu.VMEM((_PHASE,), jnp.float32),
            pltpu.SemaphoreType.DMA,
        ],
    )(_sc_body)
    return run(s1, r1, o1, e_table, r_table)


# speedup vs baseline: 1.9450x; 1.3591x over previous
import functools

import jax
import jax.numpy as jnp
from jax import lax
from jax.experimental import pallas as pl
from jax.experimental.pallas import tpu as pltpu
from jax.experimental.pallas import tpu_sc as plsc

_EMBED_DIM = 64
_BATCH = 16384
_GRP = 8

_INFO = plsc.get_sparse_core_info()
_NC, _NS, _L = _INFO.num_cores, _INFO.num_subcores, _INFO.num_lanes
_NW = _NC * _NS
_BPW = _BATCH // _NW
_PHASE = 32
_NG = _PHASE // _L


def _rsqrt_vec(x):
    i = plsc.bitcast(x, jnp.int32)
    i = jnp.int32(0x5F3759DF) - (i >> 1)
    y = plsc.bitcast(i, jnp.float32)
    hx = x * jnp.float32(-0.5)
    c = jnp.float32(1.5)
    y = y * (c + hx * y * y)
    y = y * (c + hx * y * y)
    y = y * (c + hx * y * y)
    return y


def _sc_body(s_hbm, r_hbm, o_hbm, e_hbm, rt_hbm, out_hbm,
             idx_s, idx_r, idx_o, se_v, re_v, oe_v, out_v, sem):
    wid = lax.axis_index("s") * _NC + lax.axis_index("c")
    base = wid * _BPW

    eps = jnp.float32(1e-24)
    iota = lax.iota(jnp.int32, _L)
    zero = jnp.zeros((_L,), jnp.float32)
    seven = jnp.full((_L,), _GRP - 1, jnp.int32)

    for ph in range(_BPW // _PHASE):
        pbase = base + ph * _PHASE

        pltpu.sync_copy(s_hbm.at[pl.ds(pbase, _PHASE)], idx_s)
        pltpu.sync_copy(r_hbm.at[pl.ds(pbase, _PHASE)], idx_r)
        pltpu.sync_copy(o_hbm.at[pl.ds(pbase, _PHASE)], idx_o)

        def fetch(k, carry):
            v_s = idx_s[pl.ds(k * _L, _L)]
            v_r = idx_r[pl.ds(k * _L, _L)]
            v_o = idx_o[pl.ds(k * _L, _L)]
            for t in range(_L):
                slot = k * _L + t
                dst = pl.ds(pl.multiple_of(slot * _GRP, _GRP), _GRP)
                pltpu.async_copy(e_hbm.at[v_s[t] >> 3], se_v.at[dst, :], sem)
                pltpu.async_copy(rt_hbm.at[v_r[t] >> 3], re_v.at[dst, :], sem)
                pltpu.async_copy(e_hbm.at[v_o[t] >> 3], oe_v.at[dst, :], sem)
            return carry

        lax.fori_loop(0, _NG, fetch, 0)

        def drain(k, carry):
            for _ in range(3):
                pltpu.make_async_copy(
                    e_hbm.at[0],
                    se_v.at[pl.ds(0, _GRP), :], sem).wait()
            return carry

        lax.fori_loop(0, _PHASE, drain, 0)

        def group(g, carry):
            gsl = pl.ds(g * _L, _L)
            slot16 = (g * _L + iota) * _GRP
            rl_s = slot16 + (idx_s[gsl] & seven)
            rl_r = slot16 + (idx_r[gsl] & seven)
            rl_o = slot16 + (idx_o[gsl] & seven)

            def norm_body(j, c):
                ss, rs, os_ = c
                cj = (iota + j) & (_EMBED_DIM - 1)
                vs = plsc.load_gather(se_v, [rl_s, cj])
                vr = plsc.load_gather(re_v, [rl_r, cj])
                vo = plsc.load_gather(oe_v, [rl_o, cj])
                return (ss + vs * vs, rs + vr * vr, os_ + vo * vo)

            ss, rs, os_ = lax.fori_loop(0, _EMBED_DIM, norm_body,
                                        (zero, zero, zero), unroll=8)

            inv_s = _rsqrt_vec(jnp.maximum(ss, eps))
            inv_r = _rsqrt_vec(jnp.maximum(rs, eps))
            inv_o = _rsqrt_vec(jnp.maximum(os_, eps))

            def score_body(j, acc):
                cj = (iota + j) & (_EMBED_DIM - 1)
                vs = plsc.load_gather(se_v, [rl_s, cj])
                vr = plsc.load_gather(re_v, [rl_r, cj])
                vo = plsc.load_gather(oe_v, [rl_o, cj])
                return acc + jnp.abs(vs * inv_s + vr * inv_r - vo * inv_o)

            acc = lax.fori_loop(0, _EMBED_DIM, score_body, zero, unroll=8)
            out_v[gsl] = acc
            return carry

        lax.fori_loop(0, _NG, group, 0)

        pltpu.sync_copy(out_v, out_hbm.at[pl.ds(pbase, _PHASE)])


@jax.jit
def kernel(s, r, o, e_table, r_table):
    e3 = e_table.reshape(e_table.shape[0] // _GRP, _GRP, _EMBED_DIM)
    rt3 = r_table.reshape(r_table.shape[0] // _GRP, _GRP, _EMBED_DIM)
    s1 = s.astype(jnp.int32)
    r1 = r.astype(jnp.int32)
    o1 = o.astype(jnp.int32)

    mesh = plsc.VectorSubcoreMesh(core_axis_name="c", subcore_axis_name="s")
    run = functools.partial(
        pl.kernel,
        mesh=mesh,
        compiler_params=pltpu.CompilerParams(needs_layout_passes=False),
        out_type=jax.ShapeDtypeStruct((_BATCH,), jnp.float32),
        scratch_types=[
            pltpu.VMEM((_PHASE,), jnp.int32),
            pltpu.VMEM((_PHASE,), jnp.int32),
            pltpu.VMEM((_PHASE,), jnp.int32),
            pltpu.VMEM((_PHASE * _GRP, _EMBED_DIM), jnp.float32),
            pltpu.VMEM((_PHASE * _GRP, _EMBED_DIM), jnp.float32),
            pltpu.VMEM((_PHASE * _GRP, _EMBED_DIM), jnp.float32),
            pltpu.VMEM((_PHASE,), jnp.float32),
            pltpu.SemaphoreType.DMA,
        ],
    )(_sc_body)
    return run(s1, r1, o1, e3, rt3)


# speedup vs baseline: 2.1250x; 1.0925x over previous
import functools

import jax
import jax.numpy as jnp
from jax import lax
from jax.experimental import pallas as pl
from jax.experimental.pallas import tpu as pltpu
from jax.experimental.pallas import tpu_sc as plsc

_EMBED_DIM = 64
_BATCH = 16384
_GRP = 8

_INFO = plsc.get_sparse_core_info()
_NC, _NS, _L = _INFO.num_cores, _INFO.num_subcores, _INFO.num_lanes
_NW = _NC * _NS
_BPW = _BATCH // _NW
_PHASE = _L
_NPH = _BPW // _PHASE
_NIT = _NPH // 2


def _rsqrt_vec(x):
    i = plsc.bitcast(x, jnp.int32)
    i = jnp.int32(0x5F3759DF) - (i >> 1)
    y = plsc.bitcast(i, jnp.float32)
    hx = x * jnp.float32(-0.5)
    c = jnp.float32(1.5)
    y = y * (c + hx * y * y)
    y = y * (c + hx * y * y)
    y = y * (c + hx * y * y)
    return y


def _sc_body(s_hbm, r_hbm, o_hbm, e_hbm, rt_hbm, out_hbm,
             ia_s, ia_r, ia_o, ib_s, ib_r, ib_o,
             sa_v, ra_v, oa_v, sb_v, rb_v, ob_v,
             out_v, sem_a, sem_b):
    wid = lax.axis_index("s") * _NC + lax.axis_index("c")
    base = wid * _BPW

    eps = jnp.float32(1e-24)
    iota = lax.iota(jnp.int32, _L)
    zero = jnp.zeros((_L,), jnp.float32)
    seven = jnp.full((_L,), _GRP - 1, jnp.int32)

    def stage_and_fetch(ph, idxs, bufs, sem):
        pbase = base + ph * _PHASE
        i_s, i_r, i_o = idxs
        se_v, re_v, oe_v = bufs
        pltpu.sync_copy(s_hbm.at[pl.ds(pbase, _PHASE)], i_s)
        pltpu.sync_copy(r_hbm.at[pl.ds(pbase, _PHASE)], i_r)
        pltpu.sync_copy(o_hbm.at[pl.ds(pbase, _PHASE)], i_o)
        v_s = i_s[pl.ds(0, _L)]
        v_r = i_r[pl.ds(0, _L)]
        v_o = i_o[pl.ds(0, _L)]
        for t in range(_L):
            dst = pl.ds(pl.multiple_of(t * _GRP, _GRP), _GRP)
            pltpu.async_copy(e_hbm.at[v_s[t] >> 3], se_v.at[dst, :], sem)
            pltpu.async_copy(rt_hbm.at[v_r[t] >> 3], re_v.at[dst, :], sem)
            pltpu.async_copy(e_hbm.at[v_o[t] >> 3], oe_v.at[dst, :], sem)

    def drain(sem, se_v):
        def body(k, carry):
            for _ in range(3):
                pltpu.make_async_copy(
                    e_hbm.at[0], se_v.at[pl.ds(0, _GRP), :], sem).wait()
            return carry
        lax.fori_loop(0, _L, body, 0)

    def compute(ph, idxs, bufs):
        pbase = base + ph * _PHASE
        i_s, i_r, i_o = idxs
        se_v, re_v, oe_v = bufs
        sl = pl.ds(0, _L)
        slot16 = iota * _GRP
        rl_s = slot16 + (i_s[sl] & seven)
        rl_r = slot16 + (i_r[sl] & seven)
        rl_o = slot16 + (i_o[sl] & seven)

        def norm_body(j, c):
            ss, rs, os_ = c
            cj = (iota + j) & (_EMBED_DIM - 1)
            vs = plsc.load_gather(se_v, [rl_s, cj])
            vr = plsc.load_gather(re_v, [rl_r, cj])
            vo = plsc.load_gather(oe_v, [rl_o, cj])
            return (ss + vs * vs, rs + vr * vr, os_ + vo * vo)

        ss, rs, os_ = lax.fori_loop(0, _EMBED_DIM, norm_body,
                                    (zero, zero, zero), unroll=8)

        inv_s = _rsqrt_vec(jnp.maximum(ss, eps))
        inv_r = _rsqrt_vec(jnp.maximum(rs, eps))
        inv_o = _rsqrt_vec(jnp.maximum(os_, eps))

        def score_body(j, acc):
            cj = (iota + j) & (_EMBED_DIM - 1)
            vs = plsc.load_gather(se_v, [rl_s, cj])
            vr = plsc.load_gather(re_v, [rl_r, cj])
            vo = plsc.load_gather(oe_v, [rl_o, cj])
            return acc + jnp.abs(vs * inv_s + vr * inv_r - vo * inv_o)

        acc = lax.fori_loop(0, _EMBED_DIM, score_body, zero, unroll=8)
        out_v[sl] = acc
        pltpu.sync_copy(out_v, out_hbm.at[pl.ds(pbase, _PHASE)])

    idxs_a = (ia_s, ia_r, ia_o)
    idxs_b = (ib_s, ib_r, ib_o)
    bufs_a = (sa_v, ra_v, oa_v)
    bufs_b = (sb_v, rb_v, ob_v)

    stage_and_fetch(0, idxs_a, bufs_a, sem_a)

    def it_body(i, carry):
        ph = i * 2
        stage_and_fetch(ph + 1, idxs_b, bufs_b, sem_b)
        drain(sem_a, sa_v)
        compute(ph, idxs_a, bufs_a)

        @pl.when(i < _NIT - 1)
        def _():
            stage_and_fetch(ph + 2, idxs_a, bufs_a, sem_a)

        drain(sem_b, sb_v)
        compute(ph + 1, idxs_b, bufs_b)
        return carry

    lax.fori_loop(0, _NIT, it_body, 0)


@jax.jit
def kernel(s, r, o, e_table, r_table):
    e3 = e_table.reshape(e_table.shape[0] // _GRP, _GRP, _EMBED_DIM)
    rt3 = r_table.reshape(r_table.shape[0] // _GRP, _GRP, _EMBED_DIM)
    s1 = s.astype(jnp.int32)
    r1 = r.astype(jnp.int32)
    o1 = o.astype(jnp.int32)

    mesh = plsc.VectorSubcoreMesh(core_axis_name="c", subcore_axis_name="s")
    rowbuf = pltpu.VMEM((_PHASE * _GRP, _EMBED_DIM), jnp.float32)
    idxbuf = pltpu.VMEM((_PHASE,), jnp.int32)
    run = functools.partial(
        pl.kernel,
        mesh=mesh,
        compiler_params=pltpu.CompilerParams(needs_layout_passes=False),
        out_type=jax.ShapeDtypeStruct((_BATCH,), jnp.float32),
        scratch_types=[
            idxbuf, idxbuf, idxbuf, idxbuf, idxbuf, idxbuf,
            rowbuf, rowbuf, rowbuf, rowbuf, rowbuf, rowbuf,
            pltpu.VMEM((_PHASE,), jnp.float32),
            pltpu.SemaphoreType.DMA,
            pltpu.SemaphoreType.DMA,
        ],
    )(_sc_body)
    return run(s1, r1, o1, e3, rt3)


# speedup vs baseline: 2.1286x; 1.0017x over previous
import functools

import jax
import jax.numpy as jnp
from jax import lax
from jax.experimental import pallas as pl
from jax.experimental.pallas import tpu as pltpu
from jax.experimental.pallas import tpu_sc as plsc

_EMBED_DIM = 64
_BATCH = 16384
_GRP = 8

_INFO = plsc.get_sparse_core_info()
_NC, _NS, _L = _INFO.num_cores, _INFO.num_subcores, _INFO.num_lanes
_NW = _NC * _NS
_BPW = _BATCH // _NW
_PHASE = _L
_NPH = _BPW // _PHASE
_NIT = _NPH // 2


def _rsqrt_vec(x):
    i = plsc.bitcast(x, jnp.int32)
    i = jnp.int32(0x5F3759DF) - (i >> 1)
    y = plsc.bitcast(i, jnp.float32)
    hx = x * jnp.float32(-0.5)
    c = jnp.float32(1.5)
    y = y * (c + hx * y * y)
    y = y * (c + hx * y * y)
    y = y * (c + hx * y * y)
    return y


def _sc_body(s_hbm, r_hbm, o_hbm, e_hbm, rt_hbm, out_hbm,
             idx_s, idx_r, idx_o,
             sa_v, ra_v, oa_v, sb_v, rb_v, ob_v,
             out_v, sem_a, sem_b):
    wid = lax.axis_index("s") * _NC + lax.axis_index("c")
    base = wid * _BPW

    eps = jnp.float32(1e-24)
    iota = lax.iota(jnp.int32, _L)
    zero = jnp.zeros((_L,), jnp.float32)
    seven = jnp.full((_L,), _GRP - 1, jnp.int32)

    pltpu.sync_copy(s_hbm.at[pl.ds(base, _BPW)], idx_s)
    pltpu.sync_copy(r_hbm.at[pl.ds(base, _BPW)], idx_r)
    pltpu.sync_copy(o_hbm.at[pl.ds(base, _BPW)], idx_o)

    def stage_and_fetch(ph, bufs, sem):
        se_v, re_v, oe_v = bufs
        psl = pl.ds(ph * _PHASE, _L)
        v_s = idx_s[psl]
        v_r = idx_r[psl]
        v_o = idx_o[psl]
        for t in range(_L):
            dst = pl.ds(pl.multiple_of(t * _GRP, _GRP), _GRP)
            pltpu.async_copy(e_hbm.at[v_s[t] >> 3], se_v.at[dst, :], sem)
            pltpu.async_copy(rt_hbm.at[v_r[t] >> 3], re_v.at[dst, :], sem)
            pltpu.async_copy(e_hbm.at[v_o[t] >> 3], oe_v.at[dst, :], sem)

    def drain(sem, se_v):
        def body(k, carry):
            for _ in range(3):
                pltpu.make_async_copy(
                    e_hbm.at[0], se_v.at[pl.ds(0, _GRP), :], sem).wait()
            return carry
        lax.fori_loop(0, _L, body, 0)

    def compute(ph, bufs):
        pbase = base + ph * _PHASE
        se_v, re_v, oe_v = bufs
        sl = pl.ds(0, _L)
        psl = pl.ds(ph * _PHASE, _L)
        slot16 = iota * _GRP
        rl_s = slot16 + (idx_s[psl] & seven)
        rl_r = slot16 + (idx_r[psl] & seven)
        rl_o = slot16 + (idx_o[psl] & seven)

        def norm_body(j, c):
            ss, rs, os_ = c
            cj = (iota + j) & (_EMBED_DIM - 1)
            vs = plsc.load_gather(se_v, [rl_s, cj])
            vr = plsc.load_gather(re_v, [rl_r, cj])
            vo = plsc.load_gather(oe_v, [rl_o, cj])
            return (ss + vs * vs, rs + vr * vr, os_ + vo * vo)

        ss, rs, os_ = lax.fori_loop(0, _EMBED_DIM, norm_body,
                                    (zero, zero, zero), unroll=8)

        inv_s = _rsqrt_vec(jnp.maximum(ss, eps))
        inv_r = _rsqrt_vec(jnp.maximum(rs, eps))
        inv_o = _rsqrt_vec(jnp.maximum(os_, eps))

        def score_body(j, acc):
            cj = (iota + j) & (_EMBED_DIM - 1)
            vs = plsc.load_gather(se_v, [rl_s, cj])
            vr = plsc.load_gather(re_v, [rl_r, cj])
            vo = plsc.load_gather(oe_v, [rl_o, cj])
            return acc + jnp.abs(vs * inv_s + vr * inv_r - vo * inv_o)

        acc = lax.fori_loop(0, _EMBED_DIM, score_body, zero, unroll=8)
        out_v[sl] = acc
        pltpu.sync_copy(out_v, out_hbm.at[pl.ds(pbase, _PHASE)])

    bufs_a = (sa_v, ra_v, oa_v)
    bufs_b = (sb_v, rb_v, ob_v)

    stage_and_fetch(0, bufs_a, sem_a)

    def it_body(i, carry):
        ph = i * 2
        stage_and_fetch(ph + 1, bufs_b, sem_b)
        drain(sem_a, sa_v)
        compute(ph, bufs_a)

        @pl.when(i < _NIT - 1)
        def _():
            stage_and_fetch(ph + 2, bufs_a, sem_a)

        drain(sem_b, sb_v)
        compute(ph + 1, bufs_b)
        return carry

    lax.fori_loop(0, _NIT, it_body, 0)


@jax.jit
def kernel(s, r, o, e_table, r_table):
    e3 = e_table.reshape(e_table.shape[0] // _GRP, _GRP, _EMBED_DIM)
    rt3 = r_table.reshape(r_table.shape[0] // _GRP, _GRP, _EMBED_DIM)
    s1 = s.astype(jnp.int32)
    r1 = r.astype(jnp.int32)
    o1 = o.astype(jnp.int32)

    mesh = plsc.VectorSubcoreMesh(core_axis_name="c", subcore_axis_name="s")
    rowbuf = pltpu.VMEM((_PHASE * _GRP, _EMBED_DIM), jnp.float32)
    idxbuf = pltpu.VMEM((_BPW,), jnp.int32)
    run = functools.partial(
        pl.kernel,
        mesh=mesh,
        compiler_params=pltpu.CompilerParams(needs_layout_passes=False),
        out_type=jax.ShapeDtypeStruct((_BATCH,), jnp.float32),
        scratch_types=[
            idxbuf, idxbuf, idxbuf,
            rowbuf, rowbuf, rowbuf, rowbuf, rowbuf, rowbuf,
            pltpu.VMEM((_PHASE,), jnp.float32),
            pltpu.SemaphoreType.DMA,
            pltpu.SemaphoreType.DMA,
        ],
    )(_sc_body)
    return run(s1, r1, o1, e3, rt3)
